# baseline (speedup 1.0000x reference)
import jax
import jax.numpy as jnp
from jax import lax
from jax.experimental import pallas as pl
from jax.experimental.pallas import tpu as pltpu, tpu_sc as plsc

NUM_FACTORS = 16
BATCH = 16384
ROWS_PER_BLOCK = 8
BLOCK = ROWS_PER_BLOCK * NUM_FACTORS
NBLOCKS = 1000000 // ROWS_PER_BLOCK
NC, NS = 2, 16
NW = NC * NS
BPW = BATCH // NW
CHUNK = 128
NCH = BPW // CHUNK


def _sc_body(users_hbm, movies_hbm, uf_hbm, mf_hbm, out_hbm,
             vidx_u, vidx_m, bix_u, bix_m, ublk, mblk, outv, sem_u, sem_m):
    wid = lax.axis_index("s") * NC + lax.axis_index("c")
    base = wid * BPW
    pltpu.sync_copy(users_hbm.at[pl.ds(base, BPW)], vidx_u)
    pltpu.sync_copy(movies_hbm.at[pl.ds(base, BPW)], vidx_m)

    def build(g, carry):
        j = g * 16
        bix_u[pl.ds(j, 16)] = vidx_u[pl.ds(j, 16)] >> 3
        bix_m[pl.ds(j, 16)] = vidx_m[pl.ds(j, 16)] >> 3
        return carry

    lax.fori_loop(0, BPW // 16, build, 0)

    def fire(c):
        sl = pl.ds(c * CHUNK, CHUNK)
        slot = c % 2
        pltpu.async_copy(uf_hbm.at[bix_u.at[sl]], ublk.at[slot], sem_u)
        pltpu.async_copy(mf_hbm.at[bix_m.at[sl]], mblk.at[slot], sem_m)

    def drain():
        pltpu.make_async_copy(uf_hbm.at[pl.ds(0, CHUNK)], ublk.at[0], sem_u).wait()
        pltpu.make_async_copy(mf_hbm.at[pl.ds(0, CHUNK)], mblk.at[0], sem_m).wait()

    iota16 = lax.broadcasted_iota(jnp.int32, (NUM_FACTORS,), 0)
    dn = lax.GatherDimensionNumbers(
        offset_dims=(), collapsed_slice_dims=(0,), start_index_map=(0,))

    def perm(v, k):
        return lax.gather(v, (iota16 ^ k)[:, None], dn, slice_sizes=(1,),
                          mode=lax.GatherScatterMode.PROMISE_IN_BOUNDS)

    def compute(c):
        slot = c % 2

        def group_body(g, carry):
            j = c * CHUNK + g * 16
            qu = (vidx_u[pl.ds(j, 16)] & 7) * NUM_FACTORS
            qm = (vidx_m[pl.ds(j, 16)] & 7) * NUM_FACTORS
            acc = jnp.zeros((NUM_FACTORS,), jnp.float32)
            for t in range(16):
                u = ublk[slot, g * 16 + t, pl.ds(qu[t], NUM_FACTORS)]
                m = mblk[slot, g * 16 + t, pl.ds(qm[t], NUM_FACTORS)]
                p = u * m
                for k in (1, 2, 4, 8):
                    p = p + perm(p, k)
                acc = jnp.where(iota16 == t, p, acc)
            outv[pl.ds(j, 16)] = acc
            return carry

        lax.fori_loop(0, CHUNK // 16, group_body, 0)

    fire(0)
    for c in range(NCH):
        if c + 1 < NCH:
            fire(c + 1)
        drain()
        compute(c)

    pltpu.sync_copy(outv, out_hbm.at[pl.ds(base, BPW)])


TW = 4096
TGRID = -(-1000000 // TW)


def _tc_transpose_body(x_ref, o_ref):
    x = x_ref[...]
    eye = jax.lax.broadcasted_iota(jnp.int32, (16, 16), 0) == \
        jax.lax.broadcasted_iota(jnp.int32, (16, 16), 1)
    o_ref[...] = jax.lax.dot_general(
        x, eye.astype(jnp.float32), (((0,), (0,)), ((), ())),
        preferred_element_type=jnp.float32,
        precision=jax.lax.Precision.HIGHEST)


def _detile(table_t):
    return pl.pallas_call(
        _tc_transpose_body,
        out_shape=jax.ShapeDtypeStruct((1000000, NUM_FACTORS), jnp.float32),
        grid=(TGRID,),
        in_specs=[pl.BlockSpec((NUM_FACTORS, TW), lambda k: (0, k))],
        out_specs=pl.BlockSpec((TW, NUM_FACTORS), lambda k: (k, 0)),
    )(table_t)


def kernel(data, user_factors, movie_factors):
    users = data[:, 0]
    movies = data[:, 1]
    uf_lin = _detile(user_factors.T)
    mf_lin = _detile(movie_factors.T)
    uf_b = uf_lin.reshape(NBLOCKS, BLOCK)
    mf_b = mf_lin.reshape(NBLOCKS, BLOCK)
    mesh = plsc.VectorSubcoreMesh(core_axis_name="c", subcore_axis_name="s",
                                  num_cores=NC, num_subcores=NS)
    f = pl.kernel(
        _sc_body,
        out_type=jax.ShapeDtypeStruct((BATCH,), jnp.float32),
        mesh=mesh,
        scratch_types=[
            pltpu.VMEM((BPW,), jnp.int32),
            pltpu.VMEM((BPW,), jnp.int32),
            pltpu.VMEM((BPW,), jnp.int32),
            pltpu.VMEM((BPW,), jnp.int32),
            pltpu.VMEM((2, CHUNK, BLOCK), jnp.float32),
            pltpu.VMEM((2, CHUNK, BLOCK), jnp.float32),
            pltpu.VMEM((BPW,), jnp.float32),
            pltpu.SemaphoreType.DMA,
            pltpu.SemaphoreType.DMA,
        ],
        compiler_params=pltpu.CompilerParams(use_tc_tiling_on_sc=True),
    )
    return f(users, movies, uf_b, mf_b)


# speedup vs baseline: 1.2954x; 1.2954x over previous
import jax
import jax.numpy as jnp
from jax import lax
from jax.experimental import pallas as pl
from jax.experimental.pallas import tpu as pltpu, tpu_sc as plsc

NUM_FACTORS = 16
BATCH = 16384
ROWS_PER_BLOCK = 8
BLOCK = ROWS_PER_BLOCK * NUM_FACTORS
NBLOCKS = 1000000 // ROWS_PER_BLOCK
NC, NS = 2, 16
NW = NC * NS
BPW = BATCH // NW
CHUNK = 128
NCH = BPW // CHUNK


def _sc_body(users_hbm, movies_hbm, uf_hbm, mf_hbm, out_hbm,
             vidx_u, vidx_m, bix_u, bix_m, ublk, mblk, outv, sem_u, sem_m):
    wid = lax.axis_index("s") * NC + lax.axis_index("c")
    base = wid * BPW
    pltpu.sync_copy(users_hbm.at[pl.ds(base, BPW)], vidx_u)
    pltpu.sync_copy(movies_hbm.at[pl.ds(base, BPW)], vidx_m)

    def build(g, carry):
        j = g * 16
        bix_u[pl.ds(j, 16)] = vidx_u[pl.ds(j, 16)] >> 3
        bix_m[pl.ds(j, 16)] = vidx_m[pl.ds(j, 16)] >> 3
        return carry

    lax.fori_loop(0, BPW // 16, build, 0)

    def fire(c):
        sl = pl.ds(c * CHUNK, CHUNK)
        slot = c % 2
        pltpu.async_copy(uf_hbm.at[bix_u.at[sl]], ublk.at[slot], sem_u)
        pltpu.async_copy(mf_hbm.at[bix_m.at[sl]], mblk.at[slot], sem_m)

    def drain():
        pltpu.make_async_copy(uf_hbm.at[pl.ds(0, CHUNK)], ublk.at[0], sem_u).wait()
        pltpu.make_async_copy(mf_hbm.at[pl.ds(0, CHUNK)], mblk.at[0], sem_m).wait()

    iota16 = lax.broadcasted_iota(jnp.int32, (NUM_FACTORS,), 0)
    dn = lax.GatherDimensionNumbers(
        offset_dims=(), collapsed_slice_dims=(0,), start_index_map=(0,))

    def perm(v, k):
        return lax.gather(v, (iota16 ^ k)[:, None], dn, slice_sizes=(1,),
                          mode=lax.GatherScatterMode.PROMISE_IN_BOUNDS)

    def compute(c):
        slot = c % 2

        def group_body(g, carry):
            j = c * CHUNK + g * 16
            qu = (vidx_u[pl.ds(j, 16)] & 7) * NUM_FACTORS
            qm = (vidx_m[pl.ds(j, 16)] & 7) * NUM_FACTORS
            acc = jnp.zeros((NUM_FACTORS,), jnp.float32)
            for t in range(16):
                u = ublk[slot, g * 16 + t, pl.ds(qu[t], NUM_FACTORS)]
                m = mblk[slot, g * 16 + t, pl.ds(qm[t], NUM_FACTORS)]
                p = u * m
                for k in (1, 2, 4, 8):
                    p = p + perm(p, k)
                acc = jnp.where(iota16 == t, p, acc)
            outv[pl.ds(j, 16)] = acc
            return carry

        lax.fori_loop(0, CHUNK // 16, group_body, 0)

    fire(0)
    for c in range(NCH):
        if c + 1 < NCH:
            fire(c + 1)
        drain()
        compute(c)

    pltpu.sync_copy(outv, out_hbm.at[pl.ds(base, BPW)])


TW = 4096
TGRID = -(-1000000 // TW)


def _tc_transpose_body(x_ref, o_ref):
    o_ref[...] = x_ref[...].T


def _detile(table_t):
    return pl.pallas_call(
        _tc_transpose_body,
        out_shape=jax.ShapeDtypeStruct((1000000, NUM_FACTORS), jnp.float32),
        grid=(TGRID,),
        in_specs=[pl.BlockSpec((NUM_FACTORS, TW), lambda k: (0, k))],
        out_specs=pl.BlockSpec((TW, NUM_FACTORS), lambda k: (k, 0)),
    )(table_t)


def kernel(data, user_factors, movie_factors):
    users = data[:, 0]
    movies = data[:, 1]
    uf_lin = _detile(user_factors.T)
    mf_lin = _detile(movie_factors.T)
    uf_b = uf_lin.reshape(NBLOCKS, BLOCK)
    mf_b = mf_lin.reshape(NBLOCKS, BLOCK)
    mesh = plsc.VectorSubcoreMesh(core_axis_name="c", subcore_axis_name="s",
                                  num_cores=NC, num_subcores=NS)
    f = pl.kernel(
        _sc_body,
        out_type=jax.ShapeDtypeStruct((BATCH,), jnp.float32),
        mesh=mesh,
        scratch_types=[
            pltpu.VMEM((BPW,), jnp.int32),
            pltpu.VMEM((BPW,), jnp.int32),
            pltpu.VMEM((BPW,), jnp.int32),
            pltpu.VMEM((BPW,), jnp.int32),
            pltpu.VMEM((2, CHUNK, BLOCK), jnp.float32),
            pltpu.VMEM((2, CHUNK, BLOCK), jnp.float32),
            pltpu.VMEM((BPW,), jnp.float32),
            pltpu.SemaphoreType.DMA,
            pltpu.SemaphoreType.DMA,
        ],
        compiler_params=pltpu.CompilerParams(use_tc_tiling_on_sc=True),
    )
    return f(users, movies, uf_b, mf_b)


# speedup vs baseline: 1.9548x; 1.5090x over previous
import jax
import jax.numpy as jnp
from jax import lax
from jax.experimental import pallas as pl
from jax.experimental.pallas import tpu as pltpu, tpu_sc as plsc

NUM_FACTORS = 16
BATCH = 16384
NC, NS = 2, 16
NW = NC * NS
BPW = BATCH // NW
CHUNK = 128
NCH = BPW // CHUNK


def _sc_body(users_hbm, movies_hbm, uf_hbm, mf_hbm, out_hbm,
             vidx_u, vidx_m, urows, mrows, outv, sem_u, sem_m):
    wid = lax.axis_index("s") * NC + lax.axis_index("c")
    base = wid * BPW
    pltpu.sync_copy(users_hbm.at[pl.ds(base, BPW)], vidx_u)
    pltpu.sync_copy(movies_hbm.at[pl.ds(base, BPW)], vidx_m)

    def fire(c):
        sl = pl.ds(c * CHUNK, CHUNK)
        slot = c % 2
        pltpu.async_copy(uf_hbm.at[vidx_u.at[sl]], urows.at[slot], sem_u)
        pltpu.async_copy(mf_hbm.at[vidx_m.at[sl]], mrows.at[slot], sem_m)

    def drain():
        pltpu.make_async_copy(uf_hbm.at[pl.ds(0, CHUNK), :], urows.at[0], sem_u).wait()
        pltpu.make_async_copy(mf_hbm.at[pl.ds(0, CHUNK), :], mrows.at[0], sem_m).wait()

    iota16 = lax.broadcasted_iota(jnp.int32, (NUM_FACTORS,), 0)
    dn = lax.GatherDimensionNumbers(
        offset_dims=(), collapsed_slice_dims=(0,), start_index_map=(0,))

    def perm(v, k):
        return lax.gather(v, (iota16 ^ k)[:, None], dn, slice_sizes=(1,),
                          mode=lax.GatherScatterMode.PROMISE_IN_BOUNDS)

    def compute(c):
        slot = c % 2

        def group_body(g, carry):
            j = g * 16
            acc = jnp.zeros((NUM_FACTORS,), jnp.float32)
            for t in range(16):
                p = urows[slot, j + t, :] * mrows[slot, j + t, :]
                for k in (1, 2, 4, 8):
                    p = p + perm(p, k)
                acc = jnp.where(iota16 == t, p, acc)
            outv[pl.ds(c * CHUNK + j, 16)] = acc
            return carry

        lax.fori_loop(0, CHUNK // 16, group_body, 0)

    fire(0)
    for c in range(NCH):
        if c + 1 < NCH:
            fire(c + 1)
        drain()
        compute(c)

    pltpu.sync_copy(outv, out_hbm.at[pl.ds(base, BPW)])


def kernel(data, user_factors, movie_factors):
    users = data[:, 0]
    movies = data[:, 1]
    mesh = plsc.VectorSubcoreMesh(core_axis_name="c", subcore_axis_name="s",
                                  num_cores=NC, num_subcores=NS)
    f = pl.kernel(
        _sc_body,
        out_type=jax.ShapeDtypeStruct((BATCH,), jnp.float32),
        mesh=mesh,
        scratch_types=[
            pltpu.VMEM((BPW,), jnp.int32),
            pltpu.VMEM((BPW,), jnp.int32),
            pltpu.VMEM((2, CHUNK, NUM_FACTORS), jnp.float32),
            pltpu.VMEM((2, CHUNK, NUM_FACTORS), jnp.float32),
            pltpu.VMEM((BPW,), jnp.float32),
            pltpu.SemaphoreType.DMA,
            pltpu.SemaphoreType.DMA,
        ],
        compiler_params=pltpu.CompilerParams(use_tc_tiling_on_sc=False),
    )
    return f(users, movies, user_factors, movie_factors)


# speedup vs baseline: 2.7050x; 1.3838x over previous
import jax
import jax.numpy as jnp
from jax import lax
from jax.experimental import pallas as pl
from jax.experimental.pallas import tpu as pltpu, tpu_sc as plsc

NUM_FACTORS = 16
BATCH = 16384
NC, NS = 2, 16
NW = NC * NS
BPW = BATCH // NW
G = 16
NG = BPW // G


def _sc_body(users_hbm, movies_hbm, uf_hbm, mf_hbm, out_hbm,
             vidx_u, vidx_m, ublk, mblk, outv, sem_u, sem_m):
    wid = lax.axis_index("s") * NC + lax.axis_index("c")
    base = wid * BPW
    pltpu.sync_copy(users_hbm.at[pl.ds(base, BPW)], vidx_u)
    pltpu.sync_copy(movies_hbm.at[pl.ds(base, BPW)], vidx_m)

    def fire(g, slot):
        j = g * G
        ivu = vidx_u[pl.ds(j, G)]
        ivm = vidx_m[pl.ds(j, G)]
        for t in range(G):
            ru = pl.multiple_of((ivu[t] >> 3) * 8, 8)
            rm = pl.multiple_of((ivm[t] >> 3) * 8, 8)
            pltpu.async_copy(uf_hbm.at[pl.ds(ru, 8), :],
                             ublk.at[slot, pl.ds(t * 8, 8), :], sem_u)
            pltpu.async_copy(mf_hbm.at[pl.ds(rm, 8), :],
                             mblk.at[slot, pl.ds(t * 8, 8), :], sem_m)

    def drain():
        pltpu.make_async_copy(uf_hbm.at[pl.ds(0, G * 8), :], ublk.at[0], sem_u).wait()
        pltpu.make_async_copy(mf_hbm.at[pl.ds(0, G * 8), :], mblk.at[0], sem_m).wait()

    iota16 = lax.broadcasted_iota(jnp.int32, (NUM_FACTORS,), 0)
    dn = lax.GatherDimensionNumbers(
        offset_dims=(), collapsed_slice_dims=(0,), start_index_map=(0,))

    def perm(v, k):
        return lax.gather(v, (iota16 ^ k)[:, None], dn, slice_sizes=(1,),
                          mode=lax.GatherScatterMode.PROMISE_IN_BOUNDS)

    def compute(g, slot):
        j = g * G
        qu = vidx_u[pl.ds(j, G)] & 7
        qm = vidx_m[pl.ds(j, G)] & 7
        acc = jnp.zeros((NUM_FACTORS,), jnp.float32)
        for t in range(G):
            u = ublk[slot, t * 8 + qu[t], :]
            m = mblk[slot, t * 8 + qm[t], :]
            p = u * m
            for k in (1, 2, 4, 8):
                p = p + perm(p, k)
            acc = jnp.where(iota16 == t, p, acc)
        outv[pl.ds(j, 16)] = acc

    def body(g, carry):
        slot = g % 2

        @pl.when(g + 1 < NG)
        def _():
            fire(g + 1, (g + 1) % 2)

        drain()
        compute(g, slot)
        return carry

    fire(0, 0)
    lax.fori_loop(0, NG, body, 0)
    pltpu.sync_copy(outv, out_hbm.at[pl.ds(base, BPW)])


def kernel(data, user_factors, movie_factors):
    users = data[:, 0]
    movies = data[:, 1]
    mesh = plsc.VectorSubcoreMesh(core_axis_name="c", subcore_axis_name="s",
                                  num_cores=NC, num_subcores=NS)
    f = pl.kernel(
        _sc_body,
        out_type=jax.ShapeDtypeStruct((BATCH,), jnp.float32),
        mesh=mesh,
        scratch_types=[
            pltpu.VMEM((BPW,), jnp.int32),
            pltpu.VMEM((BPW,), jnp.int32),
            pltpu.VMEM((2, G * 8, NUM_FACTORS), jnp.float32),
            pltpu.VMEM((2, G * 8, NUM_FACTORS), jnp.float32),
            pltpu.VMEM((BPW,), jnp.float32),
            pltpu.SemaphoreType.DMA,
            pltpu.SemaphoreType.DMA,
        ],
        compiler_params=pltpu.CompilerParams(use_tc_tiling_on_sc=True),
    )
    return f(users, movies, user_factors, movie_factors)


# speedup vs baseline: 12.8864x; 4.7640x over previous
import jax
import jax.numpy as jnp
from jax import lax
from jax.experimental import pallas as pl
from jax.experimental.pallas import tpu as pltpu, tpu_sc as plsc

NUM_FACTORS = 16
BATCH = 16384
NC, NS = 2, 16
NW = NC * NS
BPW = BATCH // NW
G = 8
NG = BPW // G


def _sc_body(users_hbm, movies_hbm, uft_hbm, mft_hbm, out_hbm,
             vidx_u, vidx_m, ublk, mblk, outv, sem_u, sem_m):
    wid = lax.axis_index("s") * NC + lax.axis_index("c")
    base = wid * BPW
    pltpu.sync_copy(users_hbm.at[pl.ds(base, BPW)], vidx_u)
    pltpu.sync_copy(movies_hbm.at[pl.ds(base, BPW)], vidx_m)

    def fire(i, half, slot):
        ivu = vidx_u[pl.ds(i * 16, 16)]
        ivm = vidx_m[pl.ds(i * 16, 16)]
        for t in range(G):
            e = half * G + t
            cu = pl.multiple_of((ivu[e] >> 7) * 128, 128)
            cm = pl.multiple_of((ivm[e] >> 7) * 128, 128)
            pltpu.async_copy(uft_hbm.at[:, pl.ds(cu, 128)],
                             ublk.at[slot, pl.ds(t * 16, 16), :], sem_u)
            pltpu.async_copy(mft_hbm.at[:, pl.ds(cm, 128)],
                             mblk.at[slot, pl.ds(t * 16, 16), :], sem_m)

    def drain():
        pltpu.make_async_copy(uft_hbm.at[:, pl.ds(0, G * 128)],
                              ublk.at[0], sem_u).wait()
        pltpu.make_async_copy(mft_hbm.at[:, pl.ds(0, G * 128)],
                              mblk.at[0], sem_m).wait()

    iota16 = lax.broadcasted_iota(jnp.int32, (NUM_FACTORS,), 0)
    dn = lax.GatherDimensionNumbers(
        offset_dims=(), collapsed_slice_dims=(0,), start_index_map=(0,))

    def perm(v, k):
        return lax.gather(v, (iota16 ^ k)[:, None], dn, slice_sizes=(1,),
                          mode=lax.GatherScatterMode.PROMISE_IN_BOUNDS)

    def compute(i, half, slot, acc):
        qu = vidx_u[pl.ds(i * 16, 16)] & 127
        qm = vidx_m[pl.ds(i * 16, 16)] & 127
        slot16 = jnp.broadcast_to(jnp.int32(slot), (16,))
        for t in range(G):
            e = half * G + t
            u = plsc.load_gather(
                ublk, [slot16, t * 16 + iota16,
                       jnp.broadcast_to(qu[e], (16,))])
            m = plsc.load_gather(
                mblk, [slot16, t * 16 + iota16,
                       jnp.broadcast_to(qm[e], (16,))])
            p = u * m
            for k in (1, 2, 4, 8):
                p = p + perm(p, k)
            acc = jnp.where(iota16 == e, p, acc)
        return acc

    NITER = BPW // 16

    def body(i, carry):
        fire(i, 1, 1)
        drain()
        acc = compute(i, 0, 0, jnp.zeros((NUM_FACTORS,), jnp.float32))

        @pl.when(i + 1 < NITER)
        def _():
            fire(i + 1, 0, 0)

        drain()
        acc = compute(i, 1, 1, acc)
        outv[pl.ds(i * 16, 16)] = acc
        return carry

    fire(0, 0, 0)
    lax.fori_loop(0, NITER, body, 0)
    pltpu.sync_copy(outv, out_hbm.at[pl.ds(base, BPW)])


def kernel(data, user_factors, movie_factors):
    users = data[:, 0]
    movies = data[:, 1]
    uf_t = user_factors.T
    mf_t = movie_factors.T
    mesh = plsc.VectorSubcoreMesh(core_axis_name="c", subcore_axis_name="s",
                                  num_cores=NC, num_subcores=NS)
    f = pl.kernel(
        _sc_body,
        out_type=jax.ShapeDtypeStruct((BATCH,), jnp.float32),
        mesh=mesh,
        scratch_types=[
            pltpu.VMEM((BPW,), jnp.int32),
            pltpu.VMEM((BPW,), jnp.int32),
            pltpu.VMEM((2, G * 16, 128), jnp.float32),
            pltpu.VMEM((2, G * 16, 128), jnp.float32),
            pltpu.VMEM((BPW,), jnp.float32),
            pltpu.SemaphoreType.DMA,
            pltpu.SemaphoreType.DMA,
        ],
        compiler_params=pltpu.CompilerParams(use_tc_tiling_on_sc=True, needs_layout_passes=False),
    )
    return f(users, movies, uf_t, mf_t)


# speedup vs baseline: 12.8910x; 1.0004x over previous
import jax
import jax.numpy as jnp
from jax import lax
from jax.experimental import pallas as pl
from jax.experimental.pallas import tpu as pltpu, tpu_sc as plsc

NUM_FACTORS = 16
BATCH = 16384
NC, NS = 2, 16
NW = NC * NS
BPW = BATCH // NW
G = 8


def _sc_body(users_hbm, movies_hbm, uft_hbm, mft_hbm, out_hbm,
             vidx_u, vidx_m, ublk, mblk, outv, sem_u, sem_m):
    wid = lax.axis_index("s") * NC + lax.axis_index("c")
    base = wid * BPW
    pltpu.sync_copy(users_hbm.at[pl.ds(base, BPW)], vidx_u)
    pltpu.sync_copy(movies_hbm.at[pl.ds(base, BPW)], vidx_m)

    def fire(i, half, slot):
        ivu = vidx_u[pl.ds(i * 16, 16)]
        ivm = vidx_m[pl.ds(i * 16, 16)]
        for t in range(G):
            e = half * G + t
            cu = pl.multiple_of((ivu[e] >> 7) * 128, 128)
            cm = pl.multiple_of((ivm[e] >> 7) * 128, 128)
            pltpu.async_copy(uft_hbm.at[:, pl.ds(cu, 128)],
                             ublk.at[slot, pl.ds(t * 16, 16), :], sem_u)
            pltpu.async_copy(mft_hbm.at[:, pl.ds(cm, 128)],
                             mblk.at[slot, pl.ds(t * 16, 16), :], sem_m)

    def drain():
        pltpu.make_async_copy(uft_hbm.at[:, pl.ds(0, G * 128)],
                              ublk.at[0], sem_u).wait()
        pltpu.make_async_copy(mft_hbm.at[:, pl.ds(0, G * 128)],
                              mblk.at[0], sem_m).wait()

    iota16 = lax.broadcasted_iota(jnp.int32, (NUM_FACTORS,), 0)
    dn = lax.GatherDimensionNumbers(
        offset_dims=(), collapsed_slice_dims=(0,), start_index_map=(0,))

    def perm(v, k):
        return lax.gather(v, (iota16 ^ k)[:, None], dn, slice_sizes=(1,),
                          mode=lax.GatherScatterMode.PROMISE_IN_BOUNDS)

    def compute(i, half, slot, acc):
        qu = vidx_u[pl.ds(i * 16, 16)] & 127
        qm = vidx_m[pl.ds(i * 16, 16)] & 127
        slot16 = jnp.broadcast_to(jnp.int32(slot), (16,))
        for t in range(G):
            e = half * G + t
            u = plsc.load_gather(
                ublk, [slot16, t * 16 + iota16,
                       jnp.broadcast_to(qu[e], (16,))])
            m = plsc.load_gather(
                mblk, [slot16, t * 16 + iota16,
                       jnp.broadcast_to(qm[e], (16,))])
            p = u * m
            for k in (1, 2, 4, 8):
                p = p + perm(p, k)
            acc = jnp.where(iota16 == e, p, acc)
        return acc

    NITER = BPW // 16

    def body(i, carry):
        fire(i, 1, 1)
        drain()
        acc = compute(i, 0, 0, jnp.zeros((NUM_FACTORS,), jnp.float32))

        @pl.when(i + 1 < NITER)
        def _():
            fire(i + 1, 0, 0)

        drain()
        acc = compute(i, 1, 1, acc)
        outv[pl.ds(i * 16, 16)] = acc
        return carry

    fire(0, 0, 0)
    lax.fori_loop(0, NITER, body, 0)
    pltpu.sync_copy(outv, out_hbm.at[pl.ds(base, BPW)])


def kernel(data, user_factors, movie_factors):
    users = data[:, 0]
    movies = data[:, 1]
    uf_t = user_factors.T
    mf_t = movie_factors.T
    mesh = plsc.VectorSubcoreMesh(core_axis_name="c", subcore_axis_name="s",
                                  num_cores=NC, num_subcores=NS)
    f = pl.kernel(
        _sc_body,
        out_type=jax.ShapeDtypeStruct((BATCH,), jnp.float32),
        mesh=mesh,
        scratch_types=[
            pltpu.VMEM((BPW,), jnp.int32),
            pltpu.VMEM((BPW,), jnp.int32),
            pltpu.VMEM((2, G * 16, 128), jnp.float32),
            pltpu.VMEM((2, G * 16, 128), jnp.float32),
            pltpu.VMEM((BPW,), jnp.float32),
            pltpu.SemaphoreType.DMA,
            pltpu.SemaphoreType.DMA,
        ],
        compiler_params=pltpu.CompilerParams(use_tc_tiling_on_sc=True, needs_layout_passes=False),
    )
    return f(users, movies, uf_t, mf_t)
